# baseline (speedup 1.0000x reference)
import functools

import jax
import jax.numpy as jnp
from jax import lax
from jax.experimental import pallas as pl
from jax.experimental.pallas import tpu as pltpu
from jax.experimental.pallas import tpu_sc as plsc

MASK_VALUE = 119.0
MASK_IDX = 0.0

B, N, D, M = 256, 128, 128, 32
NC, NS = 2, 16
NW = NC * NS
GPW = B // NW
L = 16

ROWS = B * N
MPW = GPW * M
IH, IW = MPW // 128, 128
NT = GPW
TROWS = N
NBUF = 4

_mesh = plsc.VectorSubcoreMesh(
    core_axis_name="c", subcore_axis_name="s", num_cores=NC, num_subcores=NS
)


@functools.partial(
    pl.kernel,
    out_type=jax.ShapeDtypeStruct((ROWS, D), jnp.float32),
    mesh=_mesh,
    scratch_types=[
        pltpu.VMEM((GPW, M), jnp.int32),
        pltpu.VMEM((IH, IW), jnp.int32),
        pltpu.VMEM((IH, IW, D), jnp.float32),
        [pltpu.VMEM((TROWS, D), jnp.float32) for _ in range(NBUF)],
        pltpu.SemaphoreType.DMA,
        pltpu.SemaphoreType.DMA,
        [pltpu.SemaphoreType.DMA for _ in range(NBUF)],
        [pltpu.SemaphoreType.DMA for _ in range(NBUF)],
    ],
)
def _mask_kernel(nf_hbm, idx_hbm, out_hbm, idx_raw, idx_row, rows_v,
                 bufs, gsem, ssem, sin, sout):
    wid = lax.axis_index("s") * NC + lax.axis_index("c")
    base = wid * GPW
    row0 = base * N

    cin = {}
    for t in range(NBUF):
        cin[t] = pltpu.async_copy(
            nf_hbm.at[pl.ds(row0 + t * TROWS, TROWS)], bufs[t], sin[t])

    pltpu.sync_copy(idx_hbm.at[pl.ds(base, GPW)], idx_raw)

    for g in range(GPW):
        boff = (base + g) * N
        for j in range(M // L):
            r = idx_raw[g, pl.ds(j * L, L)]
            lin = g * M + j * L
            idx_row[lin // IW, pl.ds(lin % IW, L)] = r + boff

    gets = [
        pltpu.async_copy(nf_hbm.at[idx_row.at[h]], rows_v.at[h], gsem)
        for h in range(IH)
    ]

    for cp in gets:
        cp.wait()
    lane = lax.iota(jnp.int32, L)
    head = lane < 2
    cvec = jnp.where(lane == 0, MASK_VALUE, MASK_IDX)
    for h in range(IH):
        for i in range(IW):
            v = rows_v[h, i, pl.ds(0, L)]
            rows_v[h, i, pl.ds(0, L)] = jnp.where(head, cvec, v)

    GPS = NT // IH
    cout = {}
    scats = []
    for t in range(NT):
        s = t % NBUF
        if t >= NBUF:
            tp = t - NBUF
            cout[tp].wait()
            if (tp + 1) % GPS == 0:
                h = tp // GPS
                scats.append(pltpu.async_copy(
                    rows_v.at[h], out_hbm.at[idx_row.at[h]], ssem))
            cin[t] = pltpu.async_copy(
                nf_hbm.at[pl.ds(row0 + t * TROWS, TROWS)], bufs[s], sin[s])
        cin[t].wait()
        cout[t] = pltpu.async_copy(
            bufs[s], out_hbm.at[pl.ds(row0 + t * TROWS, TROWS)], sout[s])
    for t in range(NT - NBUF, NT):
        cout[t].wait()
        if (t + 1) % GPS == 0:
            h = t // GPS
            scats.append(pltpu.async_copy(
                rows_v.at[h], out_hbm.at[idx_row.at[h]], ssem))
    for cp in scats:
        cp.wait()


def kernel(node_features, adjacency, nodes_to_mask):
    del adjacency
    flat = node_features.reshape(ROWS, D)
    return _mask_kernel(flat, nodes_to_mask).reshape(B, N, D)


# speedup vs baseline: 1.1710x; 1.1710x over previous
import functools

import jax
import jax.numpy as jnp
from jax import lax
from jax.experimental import pallas as pl
from jax.experimental.pallas import tpu as pltpu
from jax.experimental.pallas import tpu_sc as plsc

MASK_VALUE = 119.0
MASK_IDX = 0.0

B, N, D, M = 256, 128, 128, 32
NC, NS = 2, 16
NW = NC * NS
GPW = B // NW
L = 16

G = N * D
GPT = 2
NT = GPW // GPT
TSZ = GPT * G
NBUF = 3

_mesh = plsc.VectorSubcoreMesh(
    core_axis_name="c", subcore_axis_name="s", num_cores=NC, num_subcores=NS
)


@functools.partial(
    pl.kernel,
    out_type=jax.ShapeDtypeStruct((B * G,), jnp.float32),
    mesh=_mesh,
    scratch_types=[
        pltpu.VMEM((GPW, M), jnp.int32),
        [pltpu.VMEM((TSZ,), jnp.float32) for _ in range(NBUF)],
        [pltpu.SemaphoreType.DMA for _ in range(NBUF)],
        [pltpu.SemaphoreType.DMA for _ in range(NBUF)],
    ],
    compiler_params=pltpu.CompilerParams(needs_layout_passes=False),
)
def _mask_kernel(nf_hbm, idx_hbm, out_hbm, idx_raw, bufs, sin, sout):
    wid = lax.axis_index("s") * NC + lax.axis_index("c")
    base = wid * GPW
    woff = base * G

    cin = {}
    for t in range(NBUF):
        cin[t] = pltpu.async_copy(
            nf_hbm.at[pl.ds(woff + t * TSZ, TSZ)], bufs[t], sin[t])

    pltpu.sync_copy(idx_hbm.at[pl.ds(base, GPW)], idx_raw)

    v_mask = jnp.full((L,), MASK_VALUE, jnp.float32)
    v_zero = jnp.full((L,), MASK_IDX, jnp.float32)

    cout = {}
    for t in range(NT):
        s = t % NBUF
        if t >= NBUF:
            cout[t - NBUF].wait()
            cin[t] = pltpu.async_copy(
                nf_hbm.at[pl.ds(woff + t * TSZ, TSZ)], bufs[s], sin[s])
        cin[t].wait()
        for gi in range(GPT):
            g = t * GPT + gi
            for j in range(M // L):
                r = idx_raw[g, pl.ds(j * L, L)]
                flat = r * D + (gi * G)
                plsc.store_scatter(bufs[s], [flat], v_mask)
                plsc.store_scatter(bufs[s], [flat + 1], v_zero)
        cout[t] = pltpu.async_copy(
            bufs[s], out_hbm.at[pl.ds(woff + t * TSZ, TSZ)], sout[s])
    for t in range(NT - NBUF, NT):
        if t >= 0:
            cout[t].wait()


def kernel(node_features, adjacency, nodes_to_mask):
    del adjacency
    flat = node_features.reshape(B * G)
    return _mask_kernel(flat, nodes_to_mask).reshape(B, N, D)
